# baseline (speedup 1.0000x reference)
import functools

import jax
import jax.numpy as jnp
from jax import lax
from jax.experimental import pallas as pl
from jax.experimental.pallas import tpu as pltpu

_ANT_IDX = (3, 7, 12, 20)
_CONS_IDX = (45, 60, 77)
_SIGNS_A = (1.0, -1.0, 1.0, -1.0)
_SIGNS_C = (1.0, -1.0, 1.0)
_ALL_IDX = _ANT_IDX + _CONS_IDX
_MIN_W, _MAX_W = 0.0, 500.0

_BM = 2048
_COLS_READ = 128


def _delta_columns(x, w):
    t_a = [-s * x[:, c:c + 1] for s, c in zip(_SIGNS_A, _ANT_IDX)]
    m_a = functools.reduce(jnp.maximum, t_a)
    e_a = [jnp.exp(t - m_a) for t in t_a]
    z_a = functools.reduce(jnp.add, e_a)
    d_a = [-s * w * (e / z_a) for s, e in zip(_SIGNS_A, e_a)]
    t_c = [s * x[:, c:c + 1] for s, c in zip(_SIGNS_C, _CONS_IDX)]
    m_c = functools.reduce(jnp.maximum, t_c)
    e_c = [jnp.exp(t - m_c) for t in t_c]
    z_c = functools.reduce(jnp.add, e_c)
    d_c = [s * w * (e / z_c) for s, e in zip(_SIGNS_C, e_c)]
    return d_a + d_c


def _body(w_ref, x_ref, out_ref, delta_ref):
    w = jnp.clip(w_ref[0, 0], _MIN_W, _MAX_W)
    x = x_ref[...]
    d = _delta_columns(x, w)
    delta_ref[...] = jnp.concatenate(d, axis=1)
    col = lax.broadcasted_iota(jnp.int32, out_ref.shape, 1)
    out = jnp.zeros(out_ref.shape, jnp.float32)
    for j, c in enumerate(_ALL_IDX):
        out = jnp.where(col == c, d[j], out)
    out_ref[...] = out


def kernel(ground_atoms, clause_weight):
    batch, n_pred = ground_atoms.shape
    w2d = clause_weight.reshape(1, 1)
    grid = (batch // _BM,)
    out, delta = pl.pallas_call(
        _body,
        grid=grid,
        in_specs=[
            pl.BlockSpec(memory_space=pltpu.SMEM),
            pl.BlockSpec((_BM, _COLS_READ), lambda i: (i, 0)),
        ],
        out_specs=[
            pl.BlockSpec((_BM, n_pred), lambda i: (i, 0)),
            pl.BlockSpec((_BM, len(_ALL_IDX)), lambda i: (i, 0)),
        ],
        out_shape=[
            jax.ShapeDtypeStruct((batch, n_pred), jnp.float32),
            jax.ShapeDtypeStruct((batch, len(_ALL_IDX)), jnp.float32),
        ],
    )(w2d, ground_atoms)
    return (out, delta)


# speedup vs baseline: 4.9872x; 4.9872x over previous
import numpy as np

import jax
import jax.numpy as jnp
from jax.experimental import pallas as pl
from jax.experimental.pallas import tpu as pltpu

_ANT_IDX = (3, 7, 12, 20)
_CONS_IDX = (45, 60, 77)
_SIGNS_A = (1.0, -1.0, 1.0, -1.0)
_SIGNS_C = (1.0, -1.0, 1.0)
_ALL_IDX = _ANT_IDX + _CONS_IDX
_MIN_W, _MAX_W = 0.0, 500.0

_N_PRED = 128
_BM = 2048


def _consts():
    sv = np.zeros((1, _N_PRED), np.float32)
    dv = np.zeros((1, _N_PRED), np.float32)
    msk = np.zeros((1, _N_PRED), np.float32)
    for c, s in zip(_ANT_IDX, _SIGNS_A):
        sv[0, c] = -s
        dv[0, c] = -s
        msk[0, c] = 1.0
    for c, s in zip(_CONS_IDX, _SIGNS_C):
        sv[0, c] = s
        dv[0, c] = s
        msk[0, c] = 1.0
    gm = np.zeros((_N_PRED, _N_PRED), np.float32)
    for i in _ANT_IDX:
        for j in _ANT_IDX:
            gm[i, j] = 1.0
    for i in _CONS_IDX:
        for j in _CONS_IDX:
            gm[i, j] = 1.0
    pm = np.zeros((_N_PRED, _N_PRED), np.float32)
    for k, c in enumerate(_ALL_IDX):
        pm[c, k] = 1.0
    return jnp.asarray(sv), jnp.asarray(dv), jnp.asarray(msk), \
        jnp.asarray(gm), jnp.asarray(pm)


def _body(w_ref, x_ref, sv_ref, dv_ref, msk_ref, gm_ref, pm_ref,
          out_ref, delta_ref):
    w = jnp.clip(w_ref[0, 0], _MIN_W, _MAX_W)
    x = x_ref[...]
    e = jnp.exp(x * sv_ref[...]) * msk_ref[...]
    denom = jnp.dot(e, gm_ref[...], preferred_element_type=jnp.float32)
    p = e / (denom + (1.0 - msk_ref[...]))
    out = p * (dv_ref[...] * w)
    out_ref[...] = out
    d128 = jnp.dot(out, pm_ref[...], preferred_element_type=jnp.float32)
    delta_ref[...] = d128[:, :len(_ALL_IDX)]


def kernel(ground_atoms, clause_weight):
    batch, n_pred = ground_atoms.shape
    w2d = clause_weight.reshape(1, 1)
    sv, dv, msk, gm, pm = _consts()
    grid = (batch // _BM,)
    full = pl.BlockSpec((_N_PRED, _N_PRED), lambda i: (0, 0))
    row = pl.BlockSpec((1, _N_PRED), lambda i: (0, 0))
    out, delta = pl.pallas_call(
        _body,
        grid=grid,
        in_specs=[
            pl.BlockSpec(memory_space=pltpu.SMEM),
            pl.BlockSpec((_BM, n_pred), lambda i: (i, 0)),
            row, row, row, full, full,
        ],
        out_specs=[
            pl.BlockSpec((_BM, n_pred), lambda i: (i, 0)),
            pl.BlockSpec((_BM, len(_ALL_IDX)), lambda i: (i, 0)),
        ],
        out_shape=[
            jax.ShapeDtypeStruct((batch, n_pred), jnp.float32),
            jax.ShapeDtypeStruct((batch, len(_ALL_IDX)), jnp.float32),
        ],
    )(w2d, ground_atoms, sv, dv, msk, gm, pm)
    return (out, delta)
